# speedup vs baseline: 1.2063x; 1.2063x over previous
import functools

import jax
import jax.numpy as jnp
from jax.experimental import pallas as pl
from jax.experimental.pallas import tpu as pltpu


def _row_lse_kernel(emb_ref, lse_ref):
    x = emb_ref[...]
    m = jnp.max(x, axis=1, keepdims=True)
    s = jnp.sum(jnp.exp(x - m), axis=1, keepdims=True)
    lse_ref[...] = m + jnp.log(s)


def _gather_kernel(tok_ref, lab_ref, emb_ref, lse_ref, out_ref, loss_ref,
                   *, tn, unroll, v):
    base0 = pl.program_id(0) * tn
    col = jax.lax.broadcasted_iota(jnp.int32, (1, v), 1)

    def body(j, carry):
        acc_corr, acc_lse = carry
        jb = j * unroll
        for u in range(unroll):
            r = jb + u
            t = tok_ref[base0 + r]
            lbl = lab_ref[base0 + r]
            row = emb_ref[t]
            out_ref[r] = row
            acc_corr = acc_corr + jnp.where(col == lbl, row, 0.0)
            acc_lse = acc_lse + lse_ref[t]
        return acc_corr, acc_lse

    acc_corr = jnp.zeros((1, v), jnp.float32)
    acc_lse = jnp.zeros((1, 1), jnp.float32)
    acc_corr, acc_lse = jax.lax.fori_loop(0, tn // unroll, body,
                                          (acc_corr, acc_lse))
    part = acc_lse - jnp.sum(acc_corr, axis=1, keepdims=True)
    loss_ref[...] = part.reshape(1, 1, 1)


def kernel(sequences, labels, emb_table):
    B, T = sequences.shape
    V = emb_table.shape[0]
    N = B * T

    tok = sequences.reshape(N).astype(jnp.int32)
    lab = labels.reshape(N).astype(jnp.int32)
    emb = emb_table.astype(jnp.float32)

    RB = min(256, V)
    lse = pl.pallas_call(
        _row_lse_kernel,
        out_shape=jax.ShapeDtypeStruct((V, 1), jnp.float32),
        grid=(V // RB,),
        in_specs=[pl.BlockSpec((RB, V), lambda i: (i, 0))],
        out_specs=pl.BlockSpec((RB, 1), lambda i: (i, 0)),
        compiler_params=pltpu.CompilerParams(
            dimension_semantics=("parallel",)),
    )(emb)

    TN = 512
    while N % TN:
        TN //= 2
    num_tiles = N // TN
    UNROLL = 8 if TN % 8 == 0 else 1

    emb3 = emb.reshape(V, 1, V)
    lse3 = lse.reshape(V, 1, 1)

    grid_spec = pltpu.PrefetchScalarGridSpec(
        num_scalar_prefetch=2,
        grid=(num_tiles,),
        in_specs=[
            pl.BlockSpec((V, 1, V), lambda i, tok_s, lab_s: (0, 0, 0)),
            pl.BlockSpec((V, 1, 1), lambda i, tok_s, lab_s: (0, 0, 0)),
        ],
        out_specs=[
            pl.BlockSpec((TN, 1, V), lambda i, tok_s, lab_s: (i, 0, 0)),
            pl.BlockSpec((1, 1, 1), lambda i, tok_s, lab_s: (i, 0, 0)),
        ],
    )
    out, loss_parts = pl.pallas_call(
        functools.partial(_gather_kernel, tn=TN, unroll=UNROLL, v=V),
        grid_spec=grid_spec,
        out_shape=(
            jax.ShapeDtypeStruct((N, 1, V), jnp.float32),
            jax.ShapeDtypeStruct((num_tiles, 1, 1), jnp.float32),
        ),
        compiler_params=pltpu.CompilerParams(
            dimension_semantics=("parallel",),
            vmem_limit_bytes=56 * 1024 * 1024),
    )(tok, lab, emb3, lse3)

    prediction_scores = out.reshape(B, T, V)
    loss = jnp.sum(loss_parts) / N
    return prediction_scores, loss


# speedup vs baseline: 1.2985x; 1.0764x over previous
import functools

import jax
import jax.numpy as jnp
from jax.experimental import pallas as pl
from jax.experimental.pallas import tpu as pltpu


def _row_lse_kernel(emb_ref, lse_ref):
    x = emb_ref[...]
    m = jnp.max(x, axis=1, keepdims=True)
    s = jnp.sum(jnp.exp(x - m), axis=1, keepdims=True)
    lse_ref[...] = m + jnp.log(s)


def _gather_kernel(tok_ref, lab_ref, emb_ref, lse_ref, out_ref, loss_ref,
                   *, tn, unroll, sub, lane):
    base0 = pl.program_id(0) * tn
    flat = (lane * jax.lax.broadcasted_iota(jnp.int32, (sub, lane), 0)
            + jax.lax.broadcasted_iota(jnp.int32, (sub, lane), 1))

    def body(j, carry):
        acc_corr, acc_lse = carry
        jb = j * unroll
        for u in range(unroll):
            r = jb + u
            t = tok_ref[base0 + r]
            lbl = lab_ref[base0 + r]
            slab = emb_ref[t]
            out_ref[r] = slab
            acc_corr = acc_corr + jnp.where(flat == lbl, slab, 0.0)
            acc_lse = acc_lse + lse_ref[t]
        return acc_corr, acc_lse

    acc_corr = jnp.zeros((sub, lane), jnp.float32)
    acc_lse = jnp.zeros((1, 1), jnp.float32)
    acc_corr, acc_lse = jax.lax.fori_loop(0, tn // unroll, body,
                                          (acc_corr, acc_lse))
    part = acc_lse - jnp.sum(acc_corr, keepdims=True)
    loss_ref[...] = part.reshape(1, 1, 1)


def kernel(sequences, labels, emb_table):
    B, T = sequences.shape
    V = emb_table.shape[0]
    N = B * T
    LANE = 128
    SUB = V // LANE

    tok = sequences.reshape(N).astype(jnp.int32)
    lab = labels.reshape(N).astype(jnp.int32)
    emb = emb_table.astype(jnp.float32)

    RB = min(256, V)
    lse = pl.pallas_call(
        _row_lse_kernel,
        out_shape=jax.ShapeDtypeStruct((V, 1), jnp.float32),
        grid=(V // RB,),
        in_specs=[pl.BlockSpec((RB, V), lambda i: (i, 0))],
        out_specs=pl.BlockSpec((RB, 1), lambda i: (i, 0)),
        compiler_params=pltpu.CompilerParams(
            dimension_semantics=("parallel",)),
    )(emb)

    TN = 512
    while N % TN:
        TN //= 2
    num_tiles = N // TN
    UNROLL = 8 if TN % 8 == 0 else 1

    emb3 = emb.reshape(V, SUB, LANE)
    lse3 = lse.reshape(V, 1, 1)

    grid_spec = pltpu.PrefetchScalarGridSpec(
        num_scalar_prefetch=2,
        grid=(num_tiles,),
        in_specs=[
            pl.BlockSpec((V, SUB, LANE), lambda i, tok_s, lab_s: (0, 0, 0)),
            pl.BlockSpec((V, 1, 1), lambda i, tok_s, lab_s: (0, 0, 0)),
        ],
        out_specs=[
            pl.BlockSpec((TN, SUB, LANE), lambda i, tok_s, lab_s: (i, 0, 0)),
            pl.BlockSpec((1, 1, 1), lambda i, tok_s, lab_s: (i, 0, 0)),
        ],
    )
    out, loss_parts = pl.pallas_call(
        functools.partial(_gather_kernel, tn=TN, unroll=UNROLL,
                          sub=SUB, lane=LANE),
        grid_spec=grid_spec,
        out_shape=(
            jax.ShapeDtypeStruct((N, SUB, LANE), jnp.float32),
            jax.ShapeDtypeStruct((num_tiles, 1, 1), jnp.float32),
        ),
        compiler_params=pltpu.CompilerParams(
            dimension_semantics=("parallel",),
            vmem_limit_bytes=56 * 1024 * 1024),
    )(tok, lab, emb3, lse3)

    prediction_scores = out.reshape(B, T, V)
    loss = jnp.sum(loss_parts) / N
    return prediction_scores, loss


# speedup vs baseline: 1.5189x; 1.1697x over previous
import functools

import jax
import jax.numpy as jnp
from jax.experimental import pallas as pl
from jax.experimental.pallas import tpu as pltpu


def _row_lse_kernel(emb_ref, lse_ref):
    x = emb_ref[...]
    m = jnp.max(x, axis=1, keepdims=True)
    s = jnp.sum(jnp.exp(x - m), axis=1, keepdims=True)
    lse_ref[...] = m + jnp.log(s)


def _logits_loss_kernel(tok_ref, lab_ref, emb_ref, lse_ref, logits_ref,
                        loss_ref, *, v):
    tn = tok_ref.shape[0]
    col = jax.lax.broadcasted_iota(jnp.int32, (tn, v), 1)
    tok_mask = tok_ref[...] == col
    onehot = jnp.where(tok_mask, 1.0, 0.0).astype(jnp.bfloat16)
    logits = jnp.dot(onehot, emb_ref[...],
                     preferred_element_type=jnp.float32)
    logits_ref[...] = logits

    lse_row = lse_ref[0]
    picked = (jnp.where(tok_mask, lse_row, 0.0)
              - jnp.where(lab_ref[...] == col, logits, 0.0))
    loss_ref[...] = jnp.sum(picked, axis=1, keepdims=True)


def kernel(sequences, labels, emb_table):
    B, T = sequences.shape
    V = emb_table.shape[0]
    N = B * T

    tok = sequences.reshape(N, 1).astype(jnp.int32)
    lab = labels.reshape(N, 1).astype(jnp.int32)
    emb = emb_table.astype(jnp.float32)

    RB = min(256, V)
    lse = pl.pallas_call(
        _row_lse_kernel,
        out_shape=jax.ShapeDtypeStruct((V, 1), jnp.float32),
        grid=(V // RB,),
        in_specs=[pl.BlockSpec((RB, V), lambda i: (i, 0))],
        out_specs=pl.BlockSpec((RB, 1), lambda i: (i, 0)),
        compiler_params=pltpu.CompilerParams(
            dimension_semantics=("parallel",)),
    )(emb)
    lse_row = lse.reshape(1, 1, V)

    TN = 512
    while N % TN:
        TN //= 2
    num_tiles = N // TN
    emb_bf16 = emb.astype(jnp.bfloat16)

    logits, per_row_loss = pl.pallas_call(
        functools.partial(_logits_loss_kernel, v=V),
        out_shape=(
            jax.ShapeDtypeStruct((N, V), jnp.float32),
            jax.ShapeDtypeStruct((N, 1), jnp.float32),
        ),
        grid=(num_tiles,),
        in_specs=[
            pl.BlockSpec((TN, 1), lambda i: (i, 0)),
            pl.BlockSpec((TN, 1), lambda i: (i, 0)),
            pl.BlockSpec((V, V), lambda i: (0, 0)),
            pl.BlockSpec((1, 1, V), lambda i: (0, 0, 0)),
        ],
        out_specs=(
            pl.BlockSpec((TN, V), lambda i: (i, 0)),
            pl.BlockSpec((TN, 1), lambda i: (i, 0)),
        ),
        compiler_params=pltpu.CompilerParams(
            dimension_semantics=("parallel",),
            vmem_limit_bytes=56 * 1024 * 1024),
    )(tok, lab, emb_bf16, lse_row)

    prediction_scores = logits.reshape(B, T, V)
    loss = jnp.sum(per_row_loss) / N
    return prediction_scores, loss


# speedup vs baseline: 1.6466x; 1.0841x over previous
import functools

import jax
import jax.numpy as jnp
from jax.experimental import pallas as pl
from jax.experimental.pallas import tpu as pltpu


def _row_lse_kernel(emb_ref, lse_ref):
    x = emb_ref[...]
    m = jnp.max(x, axis=1, keepdims=True)
    s = jnp.sum(jnp.exp(x - m), axis=1, keepdims=True)
    lse_ref[...] = m + jnp.log(s)


def _gather_mm_kernel(tok_ref, lab_ref, p_ref, emb_ref, lse_ref,
                      out_ref, loss_ref, *, tn, sub, lane):
    base0 = pl.program_id(0) * tn
    flat = (lane * jax.lax.broadcasted_iota(jnp.int32, (sub, lane), 0)
            + jax.lax.broadcasted_iota(jnp.int32, (sub, lane), 1))
    pmat = p_ref[...]

    def group(g, carry):
        acc_corr, acc_lse = carry
        row_base = g * 32
        slabs = []
        for r in range(32):
            t = tok_ref[base0 + row_base + r]
            slabs.append(emb_ref[t])
        s_a = jnp.concatenate(slabs[:16], axis=0)
        s_b = jnp.concatenate(slabs[16:], axis=0)
        s_mat = jnp.concatenate([s_a, s_b], axis=1)
        o_mat = jnp.dot(pmat, s_mat,
                        preferred_element_type=jnp.float32)
        for j in range(sub):
            for r8 in range(2):
                orow = 16 * j + 8 * r8
                dst_a = pl.multiple_of(row_base + 8 * r8, 8)
                dst_b = pl.multiple_of(row_base + 16 + 8 * r8, 8)
                out_ref[pl.ds(dst_a, 8), lane * j:lane * (j + 1)] = (
                    o_mat[orow:orow + 8, 0:lane])
                out_ref[pl.ds(dst_b, 8), lane * j:lane * (j + 1)] = (
                    o_mat[orow:orow + 8, lane:2 * lane])
        for r in range(32):
            t = tok_ref[base0 + row_base + r]
            lbl = lab_ref[base0 + row_base + r]
            slab32 = emb_ref[t].astype(jnp.float32)
            acc_corr = acc_corr + jnp.where(flat == lbl, slab32, 0.0)
            acc_lse = acc_lse + lse_ref[t]
        return acc_corr, acc_lse

    acc_corr = jnp.zeros((sub, lane), jnp.float32)
    acc_lse = jnp.zeros((1, 1), jnp.float32)
    acc_corr, acc_lse = jax.lax.fori_loop(0, tn // 32, group,
                                          (acc_corr, acc_lse))
    part = acc_lse - jnp.sum(acc_corr, keepdims=True)[:1, :1]
    loss_ref[...] = part.reshape(1, 1, 1)


def kernel(sequences, labels, emb_table):
    B, T = sequences.shape
    V = emb_table.shape[0]
    N = B * T
    LANE = 128
    SUB = V // LANE

    tok = sequences.reshape(N).astype(jnp.int32)
    lab = labels.reshape(N).astype(jnp.int32)
    emb = emb_table.astype(jnp.float32)

    RB = min(256, V)
    lse = pl.pallas_call(
        _row_lse_kernel,
        out_shape=jax.ShapeDtypeStruct((V, 1), jnp.float32),
        grid=(V // RB,),
        in_specs=[pl.BlockSpec((RB, V), lambda i: (i, 0))],
        out_specs=pl.BlockSpec((RB, 1), lambda i: (i, 0)),
        compiler_params=pltpu.CompilerParams(
            dimension_semantics=("parallel",)),
    )(emb)
    lse3 = lse.reshape(V, 1, 1)

    PM = 16 * SUB
    o_idx = jnp.arange(PM, dtype=jnp.int32)[:, None]
    k_idx = SUB * (o_idx % 16) + o_idx // 16
    pmat = (jax.lax.broadcasted_iota(jnp.int32, (PM, PM), 1)
            == k_idx).astype(jnp.bfloat16)

    emb_b3 = emb.astype(jnp.bfloat16).reshape(V, SUB, LANE)

    TN = 512
    while N % TN:
        TN //= 2
    num_tiles = N // TN

    grid_spec = pltpu.PrefetchScalarGridSpec(
        num_scalar_prefetch=2,
        grid=(num_tiles,),
        in_specs=[
            pl.BlockSpec((PM, PM), lambda i, tok_s, lab_s: (0, 0)),
            pl.BlockSpec((V, SUB, LANE), lambda i, tok_s, lab_s: (0, 0, 0)),
            pl.BlockSpec((V, 1, 1), lambda i, tok_s, lab_s: (0, 0, 0)),
        ],
        out_specs=[
            pl.BlockSpec((TN, V), lambda i, tok_s, lab_s: (i, 0)),
            pl.BlockSpec((1, 1, 1), lambda i, tok_s, lab_s: (i, 0, 0)),
        ],
    )
    logits, loss_parts = pl.pallas_call(
        functools.partial(_gather_mm_kernel, tn=TN, sub=SUB, lane=LANE),
        grid_spec=grid_spec,
        out_shape=(
            jax.ShapeDtypeStruct((N, V), jnp.float32),
            jax.ShapeDtypeStruct((num_tiles, 1, 1), jnp.float32),
        ),
        compiler_params=pltpu.CompilerParams(
            dimension_semantics=("parallel",),
            vmem_limit_bytes=56 * 1024 * 1024),
    )(tok, lab, pmat, emb_b3, lse3)

    prediction_scores = logits.reshape(B, T, V)
    loss = jnp.sum(loss_parts) / N
    return prediction_scores, loss


# speedup vs baseline: 2.2161x; 1.3458x over previous
import functools

import jax
import jax.numpy as jnp
from jax.experimental import pallas as pl
from jax.experimental.pallas import tpu as pltpu


def _row_lse_kernel(emb_ref, lse_ref):
    x = emb_ref[...]
    m = jnp.max(x, axis=1, keepdims=True)
    s = jnp.sum(jnp.exp(x - m), axis=1, keepdims=True)
    lse_ref[...] = m + jnp.log(s)


_HALVES = 4
_GROUP = 16 * _HALVES
_UNROLL_G = 2
_NACC = 4


def _gather_mm_kernel(tok_ref, lab_ref, p_ref, emb_ref, lse_ref,
                      out_ref, loss_ref, *, tn, sub, lane):
    base0 = pl.program_id(0) * tn
    flat = (lane * jax.lax.broadcasted_iota(jnp.int32, (sub, lane), 0)
            + jax.lax.broadcasted_iota(jnp.int32, (sub, lane), 1))
    pmat = p_ref[...]

    def body(gg, carry):
        accs_c, accs_l = carry
        accs_c, accs_l = list(accs_c), list(accs_l)
        o_mats, bases = [], []
        for u in range(_UNROLL_G):
            row_base = (gg * _UNROLL_G + u) * _GROUP
            bases.append(row_base)
            halves = []
            for h in range(_HALVES):
                slabs = [emb_ref[tok_ref[base0 + row_base + 16 * h + r]]
                         for r in range(16)]
                halves.append(jnp.concatenate(slabs, axis=0))
            s_mat = jnp.concatenate(halves, axis=1)
            o_mats.append(jnp.dot(pmat, s_mat,
                                  preferred_element_type=jnp.float32))
        for u in range(_UNROLL_G):
            o_mat, row_base = o_mats[u], bases[u]
            for j in range(sub):
                for h in range(_HALVES):
                    for r8 in range(2):
                        orow = 16 * j + 8 * r8
                        dst = pl.multiple_of(row_base + 16 * h + 8 * r8, 8)
                        out_ref[pl.ds(dst, 8), lane * j:lane * (j + 1)] = (
                            o_mat[orow:orow + 8, lane * h:lane * (h + 1)])
        for u in range(_UNROLL_G):
            row_base = bases[u]
            for r in range(_GROUP):
                t = tok_ref[base0 + row_base + r]
                lbl = lab_ref[base0 + row_base + r]
                slab32 = emb_ref[t].astype(jnp.float32)
                k = r % _NACC
                accs_c[k] = accs_c[k] + jnp.where(flat == lbl, slab32, 0.0)
                accs_l[k] = accs_l[k] + lse_ref[t]
        return tuple(accs_c), tuple(accs_l)

    accs_c = tuple(jnp.zeros((sub, lane), jnp.float32) for _ in range(_NACC))
    accs_l = tuple(jnp.zeros((1, 1), jnp.float32) for _ in range(_NACC))
    accs_c, accs_l = jax.lax.fori_loop(
        0, tn // (_GROUP * _UNROLL_G), body, (accs_c, accs_l))
    corr = sum(accs_c[1:], accs_c[0])
    lse_tot = sum(accs_l[1:], accs_l[0])
    part = lse_tot - jnp.sum(corr, keepdims=True)[:1, :1]
    loss_ref[...] = part.reshape(1, 1, 1)


def kernel(sequences, labels, emb_table):
    B, T = sequences.shape
    V = emb_table.shape[0]
    N = B * T
    LANE = 128
    SUB = V // LANE

    tok = sequences.reshape(N).astype(jnp.int32)
    lab = labels.reshape(N).astype(jnp.int32)
    emb = emb_table.astype(jnp.float32)

    RB = min(256, V)
    lse = pl.pallas_call(
        _row_lse_kernel,
        out_shape=jax.ShapeDtypeStruct((V, 1), jnp.float32),
        grid=(V // RB,),
        in_specs=[pl.BlockSpec((RB, V), lambda i: (i, 0))],
        out_specs=pl.BlockSpec((RB, 1), lambda i: (i, 0)),
        compiler_params=pltpu.CompilerParams(
            dimension_semantics=("parallel",)),
    )(emb)
    lse3 = lse.reshape(V, 1, 1)

    PM = 16 * SUB
    o_idx = jnp.arange(PM, dtype=jnp.int32)[:, None]
    k_idx = SUB * (o_idx % 16) + o_idx // 16
    pmat = (jax.lax.broadcasted_iota(jnp.int32, (PM, PM), 1)
            == k_idx).astype(jnp.bfloat16)

    emb_b3 = emb.astype(jnp.bfloat16).reshape(V, SUB, LANE)

    TN = 512
    while N % TN:
        TN //= 2
    num_tiles = N // TN

    grid_spec = pltpu.PrefetchScalarGridSpec(
        num_scalar_prefetch=2,
        grid=(num_tiles,),
        in_specs=[
            pl.BlockSpec((PM, PM), lambda i, tok_s, lab_s: (0, 0)),
            pl.BlockSpec((V, SUB, LANE), lambda i, tok_s, lab_s: (0, 0, 0)),
            pl.BlockSpec((V, 1, 1), lambda i, tok_s, lab_s: (0, 0, 0)),
        ],
        out_specs=[
            pl.BlockSpec((TN, V), lambda i, tok_s, lab_s: (i, 0)),
            pl.BlockSpec((1, 1, 1), lambda i, tok_s, lab_s: (i, 0, 0)),
        ],
    )
    logits, loss_parts = pl.pallas_call(
        functools.partial(_gather_mm_kernel, tn=TN, sub=SUB, lane=LANE),
        grid_spec=grid_spec,
        out_shape=(
            jax.ShapeDtypeStruct((N, V), jnp.float32),
            jax.ShapeDtypeStruct((num_tiles, 1, 1), jnp.float32),
        ),
        compiler_params=pltpu.CompilerParams(
            dimension_semantics=("parallel",),
            vmem_limit_bytes=56 * 1024 * 1024),
    )(tok, lab, pmat, emb_b3, lse3)

    prediction_scores = logits.reshape(B, T, V)
    loss = jnp.sum(loss_parts) / N
    return prediction_scores, loss


# speedup vs baseline: 2.4635x; 1.1117x over previous
import functools

import jax
import jax.numpy as jnp
from jax.experimental import pallas as pl
from jax.experimental.pallas import tpu as pltpu


def _row_lse_kernel(emb_ref, lse_ref):
    x = emb_ref[...]
    m = jnp.max(x, axis=1, keepdims=True)
    s = jnp.sum(jnp.exp(x - m), axis=1, keepdims=True)
    lse_ref[...] = m + jnp.log(s)


_HALVES = 8
_GROUP = 16 * _HALVES
_UNROLL_G = 1
_NACC = 4


def _gather_mm_kernel(tok_ref, lab_ref, p_ref, emb_ref, lse_ref,
                      out_ref, loss_ref, *, tn, sub, lane):
    base0 = pl.program_id(0) * tn
    flat = (lane * jax.lax.broadcasted_iota(jnp.int32, (sub, lane), 0)
            + jax.lax.broadcasted_iota(jnp.int32, (sub, lane), 1))
    pmat = p_ref[...]

    def body(gg, carry):
        accs_c, accs_l = carry
        accs_c, accs_l = list(accs_c), list(accs_l)
        o_mats, bases, slab_lists, tok_lists = [], [], [], []
        for u in range(_UNROLL_G):
            row_base = (gg * _UNROLL_G + u) * _GROUP
            bases.append(row_base)
            toks = [tok_ref[base0 + row_base + r] for r in range(_GROUP)]
            slabs = [emb_ref[t] for t in toks]
            tok_lists.append(toks)
            slab_lists.append(slabs)
            halves = [jnp.concatenate(slabs[16 * h:16 * (h + 1)], axis=0)
                      for h in range(_HALVES)]
            s_mat = jnp.concatenate(halves, axis=1)
            o_mats.append(jnp.dot(pmat, s_mat,
                                  preferred_element_type=jnp.float32))
        for u in range(_UNROLL_G):
            o_mat, row_base = o_mats[u], bases[u]
            for j in range(sub):
                for h in range(_HALVES):
                    for r8 in range(2):
                        orow = 16 * j + 8 * r8
                        dst = pl.multiple_of(row_base + 16 * h + 8 * r8, 8)
                        out_ref[pl.ds(dst, 8), lane * j:lane * (j + 1)] = (
                            o_mat[orow:orow + 8, lane * h:lane * (h + 1)])
        for u in range(_UNROLL_G):
            row_base = bases[u]
            for r in range(_GROUP):
                lbl = lab_ref[base0 + row_base + r]
                slab32 = slab_lists[u][r].astype(jnp.float32)
                k = r % _NACC
                accs_c[k] = accs_c[k] + jnp.where(flat == lbl, slab32, 0.0)
                accs_l[k] = accs_l[k] + lse_ref[tok_lists[u][r]]
        return tuple(accs_c), tuple(accs_l)

    accs_c = tuple(jnp.zeros((sub, lane), jnp.float32) for _ in range(_NACC))
    accs_l = tuple(jnp.zeros((1, 1), jnp.float32) for _ in range(_NACC))
    accs_c, accs_l = jax.lax.fori_loop(
        0, tn // (_GROUP * _UNROLL_G), body, (accs_c, accs_l))
    corr = sum(accs_c[1:], accs_c[0])
    lse_tot = sum(accs_l[1:], accs_l[0])
    part = lse_tot - jnp.sum(corr, keepdims=True)[:1, :1]
    loss_ref[...] = part.reshape(1, 1, 1)


def kernel(sequences, labels, emb_table):
    B, T = sequences.shape
    V = emb_table.shape[0]
    N = B * T
    LANE = 128
    SUB = V // LANE

    tok = sequences.reshape(N).astype(jnp.int32)
    lab = labels.reshape(N).astype(jnp.int32)
    emb = emb_table.astype(jnp.float32)

    RB = min(256, V)
    lse = pl.pallas_call(
        _row_lse_kernel,
        out_shape=jax.ShapeDtypeStruct((V, 1), jnp.float32),
        grid=(V // RB,),
        in_specs=[pl.BlockSpec((RB, V), lambda i: (i, 0))],
        out_specs=pl.BlockSpec((RB, 1), lambda i: (i, 0)),
        compiler_params=pltpu.CompilerParams(
            dimension_semantics=("parallel",)),
    )(emb)
    lse3 = lse.reshape(V, 1, 1)

    PM = 16 * SUB
    o_idx = jnp.arange(PM, dtype=jnp.int32)[:, None]
    k_idx = SUB * (o_idx % 16) + o_idx // 16
    pmat = (jax.lax.broadcasted_iota(jnp.int32, (PM, PM), 1)
            == k_idx).astype(jnp.bfloat16)

    emb_b3 = emb.astype(jnp.bfloat16).reshape(V, SUB, LANE)

    TN = 512
    while N % TN:
        TN //= 2
    num_tiles = N // TN

    grid_spec = pltpu.PrefetchScalarGridSpec(
        num_scalar_prefetch=2,
        grid=(num_tiles,),
        in_specs=[
            pl.BlockSpec((PM, PM), lambda i, tok_s, lab_s: (0, 0)),
            pl.BlockSpec((V, SUB, LANE), lambda i, tok_s, lab_s: (0, 0, 0)),
            pl.BlockSpec((V, 1, 1), lambda i, tok_s, lab_s: (0, 0, 0)),
        ],
        out_specs=[
            pl.BlockSpec((TN, V), lambda i, tok_s, lab_s: (i, 0)),
            pl.BlockSpec((1, 1, 1), lambda i, tok_s, lab_s: (i, 0, 0)),
        ],
    )
    logits, loss_parts = pl.pallas_call(
        functools.partial(_gather_mm_kernel, tn=TN, sub=SUB, lane=LANE),
        grid_spec=grid_spec,
        out_shape=(
            jax.ShapeDtypeStruct((N, V), jnp.float32),
            jax.ShapeDtypeStruct((num_tiles, 1, 1), jnp.float32),
        ),
        compiler_params=pltpu.CompilerParams(
            dimension_semantics=("parallel",),
            vmem_limit_bytes=56 * 1024 * 1024),
    )(tok, lab, pmat, emb_b3, lse3)

    prediction_scores = logits.reshape(B, T, V)
    loss = jnp.sum(loss_parts) / N
    return prediction_scores, loss
